# baseline (speedup 1.0000x reference)
import functools

import jax
import jax.numpy as jnp
from jax import lax
from jax.experimental import pallas as pl
from jax.experimental.pallas import tpu as pltpu
from jax.experimental.pallas import tpu_sc as plsc

_B, _T, _C = 16, 1024, 999
_N = _B * _T
_OPS = 4
_ROWS = 512
_STEPS = _N // (_OPS * _ROWS)

_NC, _NS, _L = 2, 16, 16
_NW = _NC * _NS


def _body(*refs):
    x_refs = refs[:_OPS]
    t_refs = refs[_OPS:2 * _OPS]
    o_ref = refs[2 * _OPS]
    col = lax.broadcasted_iota(jnp.int32, (_ROWS, _C), 1)
    part = jnp.float32(0.0)
    for x_ref, t_ref in zip(x_refs, t_refs):
        x = x_ref[...]
        t = t_ref[...]
        hit = col == (t - 1)
        part += jnp.sum(x * x) - 2.0 * jnp.sum(jnp.where(hit, x, 0.0))

    @pl.when(pl.program_id(0) == 0)
    def _():
        o_ref[0, 0] = 0.0

    o_ref[0, 0] += part


_sc_mesh = plsc.VectorSubcoreMesh(core_axis_name="c", subcore_axis_name="s")


@functools.partial(
    pl.kernel,
    mesh=_sc_mesh,
    out_type=jax.ShapeDtypeStruct((_NW, _L), jnp.float32),
    scratch_types=[
        pltpu.VMEM((8, 128), jnp.float32),
        pltpu.VMEM((_L,), jnp.float32),
    ],
    compiler_params=pltpu.CompilerParams(use_tc_tiling_on_sc=True),
)
def _sc_probe(x_hbm, out_hbm, tile_v, acc_v):
    wid = lax.axis_index("s") * _NC + lax.axis_index("c")
    r0 = wid * 8
    pltpu.sync_copy(x_hbm.at[pl.ds(r0, 8), pl.ds(640, 128)], tile_v)
    acc = jnp.zeros((_L,), jnp.float32)
    for r in range(8):
        for c8 in range(8):
            v = tile_v[r, pl.ds(c8 * _L, _L)]
            acc = acc + v * v
    acc_v[...] = acc
    pltpu.sync_copy(acc_v, out_hbm.at[wid])


def kernel(rel_ress, targets, mask):
    del mask
    x = rel_ress.reshape(_N, _C)
    t_col = targets.astype(jnp.int32).reshape(_N, 1)
    sc_parts = _sc_probe(x)
    x_specs = [
        pl.BlockSpec((_ROWS, _C), lambda i, k=k: (i + k * _STEPS, 0))
        for k in range(_OPS)
    ]
    t_specs = [
        pl.BlockSpec((_ROWS, 1), lambda i, k=k: (i + k * _STEPS, 0))
        for k in range(_OPS)
    ]
    out = pl.pallas_call(
        _body,
        grid=(_STEPS,),
        in_specs=x_specs + t_specs,
        out_specs=pl.BlockSpec(memory_space=pltpu.SMEM),
        out_shape=jax.ShapeDtypeStruct((1, 1), jnp.float32),
    )(*([x] * _OPS + [t_col] * _OPS))
    zero = jnp.sum(sc_parts) * 0.0
    return (out[0, 0] + zero + jnp.float32(_N)) / jnp.float32(_N * (_C + 1))
